# Initial kernel scaffold; baseline (speedup 1.0000x reference)
import jax
import jax.numpy as jnp
from jax.experimental import pallas as pl


def kernel(x, target, embed_W, conv_w, conv_b, U_w, final_w, final_b):
    raise NotImplementedError("write your pallas kernel here")


# speedup vs baseline: 2.7649x; 2.7649x over previous
import functools

import jax
import jax.numpy as jnp
from jax import lax
from jax.experimental import pallas as pl
from jax.experimental.pallas import tpu as pltpu


def _pick_yb(y):
    best = y
    for cand in range(8, 257, 8):
        if y % cand == 0:
            best = cand
    return best if y % 8 == 0 else y


def _conv_body(e_ref, w_ref, b_ref, h_ref, *, L, K):
    F = w_ref.shape[2]
    acc = jnp.zeros((L, F), jnp.float32)
    for k in range(K):
        acc = acc + lax.dot_general(
            e_ref[0, k:k + L, :], w_ref[k],
            (((1,), (0,)), ((), ())),
            preferred_element_type=jnp.float32)
    h_ref[0] = jnp.tanh(acc + b_ref[0][None, :])


def _attn_body(h_ref, u_ref, fw_ref, fb_ref, t_ref, alpha_ref, yhat_ref,
               lsum_ref):
    h = h_ref[0]
    u = u_ref[...]
    scores = lax.dot_general(u, h, (((1,), (1,)), ((), ())),
                             preferred_element_type=jnp.float32)
    smax = jnp.max(scores, axis=1, keepdims=True)
    p = jnp.exp(scores - smax)
    ssum = jnp.sum(p, axis=1, keepdims=True)
    alpha = p / ssum
    alpha_ref[0] = alpha
    m = lax.dot_general(alpha, h, (((1,), (0,)), ((), ())),
                        preferred_element_type=jnp.float32)
    yh = jnp.sum(fw_ref[...] * m, axis=1)[None, :] + fb_ref[0]
    yhat_ref[0, 0] = yh
    t = t_ref[0, 0]
    bce = jnp.maximum(yh, 0.0) - yh * t + jnp.log1p(jnp.exp(-jnp.abs(yh)))
    lsum_ref[0, 0] = jnp.sum(bce, axis=1, keepdims=True)


def kernel(x, target, embed_W, conv_w, conv_b, U_w, final_w, final_b):
    B, L = x.shape
    D = embed_W.shape[1]
    F, _, K = conv_w.shape
    Y = U_w.shape[0]
    PAD = K // 2
    YB = _pick_yb(Y)
    NY = Y // YB

    e = embed_W[x]
    e_pad = jnp.pad(e, ((0, 0), (PAD, PAD), (0, 0)))
    w_t = jnp.transpose(conv_w, (2, 1, 0))

    conv_fn = pl.pallas_call(
        functools.partial(_conv_body, L=L, K=K),
        grid=(B,),
        in_specs=[
            pl.BlockSpec((1, L + K - 1, D), lambda b: (b, 0, 0)),
            pl.BlockSpec((K, D, F), lambda b: (0, 0, 0)),
            pl.BlockSpec((1, F), lambda b: (0, 0)),
        ],
        out_specs=pl.BlockSpec((1, L, F), lambda b: (b, 0, 0)),
        out_shape=jax.ShapeDtypeStruct((B, L, F), jnp.float32),
        compiler_params=pltpu.CompilerParams(
            dimension_semantics=("parallel",)),
    )
    h = conv_fn(e_pad, w_t, conv_b.reshape(1, F))

    attn_fn = pl.pallas_call(
        _attn_body,
        grid=(B, NY),
        in_specs=[
            pl.BlockSpec((1, L, F), lambda b, y: (b, 0, 0)),
            pl.BlockSpec((YB, F), lambda b, y: (y, 0)),
            pl.BlockSpec((YB, F), lambda b, y: (y, 0)),
            pl.BlockSpec((1, 1, YB), lambda b, y: (y, 0, 0)),
            pl.BlockSpec((1, 1, 1, YB), lambda b, y: (b, y, 0, 0)),
        ],
        out_specs=[
            pl.BlockSpec((1, YB, L), lambda b, y: (b, y, 0)),
            pl.BlockSpec((1, 1, 1, YB), lambda b, y: (b, y, 0, 0)),
            pl.BlockSpec((1, 1, 1, 1), lambda b, y: (b, y, 0, 0)),
        ],
        out_shape=[
            jax.ShapeDtypeStruct((B, Y, L), jnp.float32),
            jax.ShapeDtypeStruct((B, NY, 1, YB), jnp.float32),
            jax.ShapeDtypeStruct((B, NY, 1, 1), jnp.float32),
        ],
        compiler_params=pltpu.CompilerParams(
            dimension_semantics=("parallel", "arbitrary")),
    )
    alpha, yhat4, lsums = attn_fn(h, U_w, final_w,
                                  final_b.reshape(NY, 1, YB),
                                  target.reshape(B, NY, 1, YB))
    yhat = yhat4.reshape(B, Y)
    loss = jnp.sum(lsums) / (B * Y)
    return yhat, loss, alpha
